# baseline (speedup 1.0000x reference)
import functools

import jax
import jax.numpy as jnp
from jax import lax
from jax.experimental import pallas as pl
from jax.experimental.pallas import tpu as pltpu
from jax.experimental.pallas import tpu_sc as plsc

_N_ROWS = 100000
_D = 64
_B = 16384
_NC = 2
_NS = 16
_NW = _NC * _NS
_BPW = _B // _NW
_CH = 256
_CB = 2048
_GRID = (_N_ROWS + _CB - 1) // _CB
_TAU = 0.5


def _inv_body(u_ref, um_ref, i_ref, im_ref, acc_ref, up_ref, ip_ref):
    step = pl.program_id(0)

    @pl.when(step == 0)
    def _init():
        acc_ref[0, 0] = 0.0

    mask = (step * _CB + lax.iota(jnp.int32, _CB)) < _N_ROWS

    def pair(a, b):
        num = jnp.sum(a * b, axis=0)
        den = (jnp.sqrt(jnp.sum(a * a, axis=0))
               * jnp.sqrt(jnp.sum(b * b, axis=0)) + 1e-10)
        return jnp.sum(jnp.where(mask, num / den, 0.0))

    um = um_ref[...]
    im = im_ref[...]
    acc_ref[0, 0] += pair(u_ref[...], um) + pair(i_ref[...], im)
    umt = um.T
    imt = im.T
    h = _CB // 2
    up_ref[...] = jnp.concatenate([umt[:h], umt[h:]], axis=1)
    ip_ref[...] = jnp.concatenate([imt[:h], imt[h:]], axis=1)


def _inv_repack(u_t, um_t, i_t, im_t):
    spec = pl.BlockSpec((_D, _CB), lambda i: (0, i))
    pack_shape = jax.ShapeDtypeStruct((_GRID * _CB // 2, 2 * _D), jnp.float32)
    return pl.pallas_call(
        _inv_body,
        grid=(_GRID,),
        in_specs=[spec, spec, spec, spec],
        out_specs=[
            pl.BlockSpec((1, 1), lambda i: (0, 0), memory_space=pltpu.SMEM),
            pl.BlockSpec((_CB // 2, 2 * _D), lambda i: (i, 0)),
            pl.BlockSpec((_CB // 2, 2 * _D), lambda i: (i, 0)),
        ],
        out_shape=[jax.ShapeDtypeStruct((1, 1), jnp.float32),
                   pack_shape, pack_shape],
    )(u_t, um_t, i_t, im_t)


def _sc_scores_body(up_hbm, ip_hbm, users_hbm, pos_hbm, neg_hbm, out_hbm,
                    uidx_v, pidx_v, nidx_v, upair_v, ppair_v, npair_v,
                    u_v, p_v, n_v, diff_v, sem_u, sem_p, sem_n):
    wid = lax.axis_index("s") * _NC + lax.axis_index("c")
    base = wid * _BPW
    pltpu.sync_copy(users_hbm.at[pl.ds(base, _BPW)], uidx_v)
    pltpu.sync_copy(pos_hbm.at[pl.ds(base, _BPW)], pidx_v)
    pltpu.sync_copy(neg_hbm.at[pl.ds(base, _BPW)], nidx_v)

    def to_line(k, _):
        for src, dst in ((uidx_v, upair_v), (pidx_v, ppair_v),
                         (nidx_v, npair_v)):
            v = src[pl.ds(k * 16, 16)]
            dst[pl.ds(k * 16, 16)] = (
                lax.shift_left(lax.shift_right_logical(v, 11), 10)
                + (v & 1023))
        return 0

    lax.fori_loop(0, _BPW // 16, to_line, 0)

    lanes = lax.iota(jnp.int32, 16)

    for ch in range(_BPW // _CH):
        off = ch * _CH
        cu = pltpu.async_copy(up_hbm.at[upair_v.at[pl.ds(off, _CH)]],
                              u_v, sem_u)
        cp = pltpu.async_copy(ip_hbm.at[ppair_v.at[pl.ds(off, _CH)]],
                              p_v, sem_p)
        cn = pltpu.async_copy(ip_hbm.at[npair_v.at[pl.ds(off, _CH)]],
                              n_v, sem_n)
        cu.wait()
        cp.wait()
        cn.wait()

        def group_body(g, _):
            gl = off + g * 16
            rows = g * 16 + lanes
            uoff = (lax.shift_right_logical(uidx_v[pl.ds(gl, 16)], 10) & 1) * _D
            poff = (lax.shift_right_logical(pidx_v[pl.ds(gl, 16)], 10) & 1) * _D
            noff = (lax.shift_right_logical(nidx_v[pl.ds(gl, 16)], 10) & 1) * _D
            acc = jnp.zeros((16,), jnp.float32)
            for dd in range(_D):
                uc = plsc.load_gather(u_v, [rows, uoff + dd])
                pc = plsc.load_gather(p_v, [rows, poff + dd])
                nc = plsc.load_gather(n_v, [rows, noff + dd])
                acc = acc + uc * (pc - nc)
            diff_v[pl.ds(gl, 16)] = acc
            return 0

        lax.fori_loop(0, _CH // 16, group_body, 0)

    pltpu.sync_copy(diff_v, out_hbm.at[pl.ds(base, _BPW)])


@functools.cache
def _sc_scores():
    return pl.kernel(
        _sc_scores_body,
        out_type=jax.ShapeDtypeStruct((_B,), jnp.float32),
        mesh=plsc.VectorSubcoreMesh(core_axis_name="c", subcore_axis_name="s"),
        compiler_params=pltpu.CompilerParams(needs_layout_passes=False),
        scratch_types=[
        pltpu.VMEM((_BPW,), jnp.int32),
        pltpu.VMEM((_BPW,), jnp.int32),
        pltpu.VMEM((_BPW,), jnp.int32),
        pltpu.VMEM((_BPW,), jnp.int32),
        pltpu.VMEM((_BPW,), jnp.int32),
        pltpu.VMEM((_BPW,), jnp.int32),
        pltpu.VMEM((_CH, 2 * _D), jnp.float32),
        pltpu.VMEM((_CH, 2 * _D), jnp.float32),
        pltpu.VMEM((_CH, 2 * _D), jnp.float32),
        pltpu.VMEM((_BPW,), jnp.float32),
        pltpu.SemaphoreType.DMA,
        pltpu.SemaphoreType.DMA,
        pltpu.SemaphoreType.DMA,
        ],
    )


def _final_body(inv_ref, d_ref, out_ref):
    x = d_ref[...]
    maxi = jnp.log(jax.nn.sigmoid(x) + 1e-10)
    mf = -jnp.sum(maxi) / _B
    inv = inv_ref[0, 0] / _N_ROWS
    out_ref[0, 0] = -inv + _TAU * mf


def _combine(inv_sum, diff):
    return pl.pallas_call(
        _final_body,
        in_specs=[pl.BlockSpec(memory_space=pltpu.SMEM),
                  pl.BlockSpec((128, 128), lambda: (0, 0))],
        out_specs=pl.BlockSpec(memory_space=pltpu.SMEM),
        out_shape=jax.ShapeDtypeStruct((1, 1), jnp.float32),
    )(inv_sum, diff.reshape(128, 128))


def kernel(all_users, all_items, all_users_m, all_items_m,
           users, pos_items, neg_items):
    users = users.astype(jnp.int32)
    pos_items = pos_items.astype(jnp.int32)
    neg_items = neg_items.astype(jnp.int32)
    inv_sum, upk, ipk = _inv_repack(all_users.T, all_users_m.T,
                                    all_items.T, all_items_m.T)
    diff = _sc_scores()(upk, ipk, users, pos_items, neg_items)
    out = _combine(inv_sum, diff)
    return out[0, 0]
